# Initial kernel scaffold; baseline (speedup 1.0000x reference)
import jax
import jax.numpy as jnp
from jax.experimental import pallas as pl


def kernel(x, offsets, emb_table, W, b):
    raise NotImplementedError("write your pallas kernel here")


# speedup vs baseline: 1.1740x; 1.1740x over previous
import functools

import jax
import jax.numpy as jnp
from jax import lax
from jax.experimental import pallas as pl
from jax.experimental.pallas import tpu as pltpu
from jax.experimental.pallas import tpu_sc as plsc

BATCH = 16384
N_FIELDS = 26
EMBED_DIM = 16

NC = 2
NS = 16
NW = NC * NS
B_PER_W = BATCH // NW
ITEMS_PER_CHUNK = 4
ROWS_PER_CHUNK = ITEMS_PER_CHUNK * N_FIELDS
NCHUNKS = B_PER_W // ITEMS_PER_CHUNK
NBUF = 4
FLAT_PER_W = B_PER_W * N_FIELDS
OFF_TILE = 208
NVEC_OFF = OFF_TILE // EMBED_DIM
NPERIODS = FLAT_PER_W // OFF_TILE


def _sc_kernel(x_hbm, off_hbm, table_hbm, w_hbm, bias_hbm, out_hbm,
               idx_v, off_v, w_v, b_v, acc_v,
               buf0, buf1, buf2, buf3,
               sem0, sem1, sem2, sem3, sem_in):
    bufs = [buf0, buf1, buf2, buf3]
    sems = [sem0, sem1, sem2, sem3]

    wid = lax.axis_index("s") * NC + lax.axis_index("c")
    base_flat = wid * FLAT_PER_W
    base_out = wid * B_PER_W

    pltpu.sync_copy(x_hbm.at[pl.ds(base_flat, FLAT_PER_W)], idx_v)
    pltpu.sync_copy(off_hbm, off_v)
    pltpu.sync_copy(w_hbm, w_v)
    pltpu.sync_copy(bias_hbm, b_v)

    off_regs = [off_v[pl.ds(k * EMBED_DIM, EMBED_DIM)] for k in range(NVEC_OFF)]

    def off_body(j, carry):
        p = j * OFF_TILE
        for k in range(NVEC_OFF):
            sl = pl.ds(p + k * EMBED_DIM, EMBED_DIM)
            idx_v[sl] = idx_v[sl] + off_regs[k]
        return carry

    lax.fori_loop(0, NPERIODS, off_body, 0)

    wv = w_v[...] * (1.0 / N_FIELDS)

    def gather_start(c, slot):
        idx_sl = idx_v.at[pl.ds(c * ROWS_PER_CHUNK, ROWS_PER_CHUNK)]
        pltpu.async_copy(table_hbm.at[idx_sl], bufs[slot], sems[slot])

    def gather_wait(slot):
        idx_sl = idx_v.at[pl.ds(0, ROWS_PER_CHUNK)]
        pltpu.make_async_copy(table_hbm.at[idx_sl], bufs[slot], sems[slot]).wait()

    for b in range(NBUF):
        gather_start(b, b)

    lane_iota = lax.iota(jnp.int32, EMBED_DIM)

    def outer(c0, carry):
        acc_vec = jnp.zeros((EMBED_DIM,), jnp.float32)
        for b in range(NBUF):
            c = c0 * NBUF + b
            gather_wait(b)
            for item in range(ITEMS_PER_CHUNK):
                acc = bufs[b][item * N_FIELDS, :] * wv
                for f in range(1, N_FIELDS):
                    acc = acc + bufs[b][item * N_FIELDS + f, :] * wv
                lane = b * ITEMS_PER_CHUNK + item
                acc_vec = jnp.where(lane_iota == lane, jnp.sum(acc), acc_vec)

            @pl.when(c + NBUF < NCHUNKS)
            def _():
                gather_start(c + NBUF, b)
        acc_v[pl.ds(c0 * EMBED_DIM, EMBED_DIM)] = acc_vec
        return carry

    lax.fori_loop(0, NCHUNKS // NBUF, outer, 0)

    bv = b_v[...]

    def sig_body(v, carry):
        sl = pl.ds(v * EMBED_DIM, EMBED_DIM)
        z = acc_v[sl] + bv
        acc_v[sl] = 1.0 / (1.0 + jnp.exp(-z))
        return carry

    lax.fori_loop(0, B_PER_W // EMBED_DIM, sig_body, 0)

    pltpu.sync_copy(acc_v, out_hbm.at[pl.ds(base_out, B_PER_W)])


@jax.jit
def kernel(x, offsets, emb_table, W, b):
    x_flat = x.astype(jnp.int32).reshape(-1)
    off_tile = jnp.tile(offsets.astype(jnp.int32), OFF_TILE // N_FIELDS)
    w_vec = W.astype(jnp.float32).reshape(EMBED_DIM)
    b_vec = jnp.broadcast_to(b.astype(jnp.float32), (EMBED_DIM,))

    mesh = plsc.VectorSubcoreMesh(core_axis_name="c", subcore_axis_name="s")
    run = pl.kernel(
        _sc_kernel,
        mesh=mesh,
        out_type=jax.ShapeDtypeStruct((BATCH,), jnp.float32),
        compiler_params=pltpu.CompilerParams(
            needs_layout_passes=False, use_tc_tiling_on_sc=False
        ),
        scratch_types=[
            pltpu.VMEM((FLAT_PER_W,), jnp.int32),
            pltpu.VMEM((OFF_TILE,), jnp.int32),
            pltpu.VMEM((EMBED_DIM,), jnp.float32),
            pltpu.VMEM((EMBED_DIM,), jnp.float32),
            pltpu.VMEM((B_PER_W,), jnp.float32),
            pltpu.VMEM((ROWS_PER_CHUNK, EMBED_DIM), jnp.float32),
            pltpu.VMEM((ROWS_PER_CHUNK, EMBED_DIM), jnp.float32),
            pltpu.VMEM((ROWS_PER_CHUNK, EMBED_DIM), jnp.float32),
            pltpu.VMEM((ROWS_PER_CHUNK, EMBED_DIM), jnp.float32),
            pltpu.SemaphoreType.DMA,
            pltpu.SemaphoreType.DMA,
            pltpu.SemaphoreType.DMA,
            pltpu.SemaphoreType.DMA,
            pltpu.SemaphoreType.DMA,
        ],
    )
    return run(x_flat, off_tile, emb_table, w_vec, b_vec)
